# baseline (speedup 1.0000x reference)
import functools

import jax
import jax.numpy as jnp
from jax import lax
from jax.experimental import pallas as pl
from jax.experimental.pallas import tpu as pltpu
from jax.experimental.pallas import tpu_sc as plsc

SUPPORT_RANGE = 300
EPS = 0.001
NUM_BINS = 2 * SUPPORT_RANGE + 1

N_ROWS = 16384
BLOCK_ROWS = 4096

NUM_CORES = 2
NUM_SUBCORES = 16
NUM_WORKERS = NUM_CORES * NUM_SUBCORES
ROWS_PER_WORKER = N_ROWS // NUM_WORKERS
CHUNK = 128
N_CHUNKS = ROWS_PER_WORKER // CHUNK
GROUPS = CHUNK // 16
FULL16 = NUM_BINS // 16


def _row_scalar_block(logits_ref, y_ref):
    logits = logits_ref[...]
    rows = logits.shape[0]

    bins = jax.lax.broadcasted_iota(jnp.int32, (rows, NUM_BINS), 1)
    support = bins.astype(jnp.float32) - float(SUPPORT_RANGE)

    m = jnp.max(logits, axis=-1, keepdims=True)
    e = jnp.exp(logits - m)
    x = jnp.sum(e * support, axis=-1, keepdims=True) / jnp.sum(
        e, axis=-1, keepdims=True
    )

    scalar = jnp.sign(x) * (
        ((jnp.sqrt(1.0 + 4.0 * EPS * (jnp.abs(x) + 1.0 + EPS)) - 1.0) / (2.0 * EPS))
        ** 2
        - 1.0
    )
    y = jnp.sign(scalar) * (jnp.sqrt(jnp.abs(scalar) + 1.0) - 1.0) + EPS * scalar
    y = jnp.clip(y, -float(SUPPORT_RANGE), float(SUPPORT_RANGE))
    y_ref[...] = y.reshape(rows // 128, 128)


def _zero_fill(out_v):
    zeros = jnp.zeros((16,), jnp.float32)

    def zero_row(r, carry):
        for u in range(FULL16):
            out_v[r, pl.ds(u * 16, 16)] = zeros
        out_v[r, pl.ds(NUM_BINS - 16, 16)] = zeros
        return carry

    lax.fori_loop(0, CHUNK, zero_row, 0)


def _sc_scatter_body(y_hbm, out_hbm, y_v, out_v):
    wid = lax.axis_index("s") * NUM_CORES + lax.axis_index("c")
    row_base = wid * ROWS_PER_WORKER

    lane = lax.iota(jnp.int32, 16)
    zeros = jnp.zeros((16,), jnp.float32)

    _zero_fill(out_v)

    def chunk_body(ci, carry):
        yrow = (row_base + ci * CHUNK) // 128
        pltpu.sync_copy(y_hbm.at[pl.ds(yrow, 1)], y_v)

        touched = []
        for g in range(GROUPS):
            rid = g * 16 + lane
            y = y_v[0, pl.ds(g * 16, 16)]
            t = y + float(SUPPORT_RANGE)
            ti = jnp.clip(t.astype(jnp.int32), 0, NUM_BINS - 1)
            frac = t - ti.astype(jnp.float32)
            ihigh = jnp.minimum(ti + 1, NUM_BINS - 1)
            plsc.store_scatter(out_v, [rid, ti], 1.0 - frac)
            plsc.addupdate_scatter(out_v, [rid, ihigh], frac)
            touched.append((rid, ti, ihigh))

        pltpu.sync_copy(out_v, out_hbm.at[pl.ds(row_base + ci * CHUNK, CHUNK)])

        for rid, ti, ihigh in touched:
            plsc.store_scatter(out_v, [rid, ti], zeros)
            plsc.store_scatter(out_v, [rid, ihigh], zeros)
        return carry

    lax.fori_loop(0, N_CHUNKS, chunk_body, 0)


@jax.jit
def kernel(logits):
    y = pl.pallas_call(
        _row_scalar_block,
        grid=(N_ROWS // BLOCK_ROWS,),
        in_specs=[pl.BlockSpec((BLOCK_ROWS, NUM_BINS), lambda i: (i, 0))],
        out_specs=pl.BlockSpec((BLOCK_ROWS // 128, 128), lambda i: (i, 0)),
        out_shape=jax.ShapeDtypeStruct((N_ROWS // 128, 128), jnp.float32),
    )(logits)

    mesh = plsc.VectorSubcoreMesh(core_axis_name="c", subcore_axis_name="s")
    scatter = functools.partial(
        pl.kernel,
        mesh=mesh,
        out_type=jax.ShapeDtypeStruct((N_ROWS, NUM_BINS), jnp.float32),
        scratch_types=[
            pltpu.VMEM((1, CHUNK), jnp.float32),
            pltpu.VMEM((CHUNK, NUM_BINS), jnp.float32),
        ],
        compiler_params=pltpu.CompilerParams(needs_layout_passes=False),
    )(_sc_scatter_body)
    return scatter(y)
